# baseline (speedup 1.0000x reference)
import functools

import jax
import jax.numpy as jnp
from jax import lax
from jax.experimental import pallas as pl
from jax.experimental.pallas import tpu as pltpu
from jax.experimental.pallas import tpu_sc as plsc

MAXP = 100000
D = 64
LANES = 16
NBUF = 4
AHEAD = 3
NS = 200
BT = 128
ST = 25


def _sc_gather(t4, pos_enc):
    mesh = plsc.VectorSubcoreMesh(core_axis_name="c", subcore_axis_name="s")

    @functools.partial(
        pl.kernel,
        out_type=jax.ShapeDtypeStruct((NS, D // 8, 32, 8 * BT), jnp.float32),
        mesh=mesh,
        scratch_types=[
            pltpu.VMEM((NS, BT), jnp.int32),
            pltpu.VMEM((NBUF, BT, D), jnp.float32),
            pltpu.VMEM((NBUF, D * BT), jnp.float32),
            pltpu.SemaphoreType.DMA,
            [pltpu.SemaphoreType.DMA] * NBUF,
            [pltpu.SemaphoreType.DMA] * NBUF,
        ],
        compiler_params=pltpu.CompilerParams(use_tc_tiling_on_sc=False,
                                             needs_layout_passes=False,
                                             disable_bounds_checks=True),
    )
    def k(t_hbm, table_hbm, out_hbm, idx_v, rows_v, tr_v,
          idx_sem, gat_sems, out_sems):
        wid = lax.axis_index("s") * 2 + lax.axis_index("c")
        iota = lax.broadcasted_iota(jnp.int32, (LANES,), 0)

        for rt in range(ST):
            pltpu.async_copy(t_hbm.at[rt, wid],
                             idx_v.at[pl.ds(rt * 8, 8)], idx_sem)
        for rt in range(ST):
            pltpu.make_async_copy(t_hbm.at[0, 0],
                                  idx_v.at[pl.ds(0, 8)], idx_sem).wait()

        def adj(r, carry):
            for c in range(BT // LANES):
                v = idx_v[r, pl.ds(c * LANES, LANES)]
                v = v - 1
                v = jnp.where(v < 0, MAXP - 1, v)
                idx_v[r, pl.ds(c * LANES, LANES)] = v
            return carry

        lax.fori_loop(0, NS, adj, 0)

        def fire_gather(s, slot):
            pltpu.async_copy(table_hbm.at[idx_v.at[s]], rows_v.at[slot],
                             gat_sems[slot])

        def drain_gather(slot):
            pltpu.make_async_copy(table_hbm.at[idx_v.at[0]], rows_v.at[0],
                                  gat_sems[slot]).wait()

        def transpose(slot):
            idx0 = iota * BT

            @plsc.parallel_loop(
                0, BT, unroll=8,
                carry=tuple(idx0 + k * LANES * BT
                            for k in range(D // LANES)))
            def tbody(b, carry):
                for k in range(D // LANES):
                    v = rows_v[slot, b, pl.ds(k * LANES, LANES)]
                    plsc.store_scatter(tr_v.at[slot], [carry[k]], v)
                return tuple(c + 1 for c in carry)

        def fire_out(s, slot):
            for dt in range(D // 8):
                pltpu.async_copy(
                    tr_v.at[slot, pl.ds(dt * 8 * BT, 8 * BT)],
                    out_hbm.at[s, dt, wid],
                    out_sems[slot])

        def drain_out(slot):
            for dt in range(D // 8):
                pltpu.make_async_copy(
                    tr_v.at[0, pl.ds(0, 8 * BT)],
                    out_hbm.at[0, 0, 0],
                    out_sems[slot]).wait()

        def step(s, slot, do_fire, do_drain_out):
            if do_drain_out:
                drain_out(slot)
            if do_fire:
                fire_gather(s + AHEAD, (slot + AHEAD) % NBUF)
            drain_gather(slot)
            transpose(slot)
            fire_out(s, slot)

        for s in range(AHEAD):
            fire_gather(s, s % NBUF)

        for s in range(NBUF):
            step(s, s % NBUF, s + AHEAD < NS, False)

        groups = (NS - 2 * NBUF) // NBUF

        def outer(i, carry):
            for b in range(NBUF):
                step(i * NBUF + NBUF + b, b, True, True)
            return carry

        lax.fori_loop(0, groups, outer, 0)

        for s in range(NBUF + groups * NBUF, NS):
            step(s, s % NBUF, s + AHEAD < NS, True)

        for s in range(NS - NBUF, NS):
            drain_out(s % NBUF)

    return k(t4, pos_enc)


def kernel(t, pos_enc):
    t4 = t.T.reshape(ST, 8, 32, BT).transpose(0, 2, 1, 3)
    out4 = _sc_gather(t4, pos_enc)
    out5 = out4.reshape(NS, D // 8, 32, 8, BT)
    return out5.transpose(2, 4, 0, 1, 3).reshape(4096, NS, D)


# speedup vs baseline: 3.4793x; 3.4793x over previous
import functools

import jax
import jax.numpy as jnp
from jax import lax
from jax.experimental import pallas as pl
from jax.experimental.pallas import tpu as pltpu
from jax.experimental.pallas import tpu_sc as plsc

MAXP = 100000
D = 64
LANES = 16
NBUF = 4
AHEAD = 3
NS = 200
BT = 128
ST = 25


def _sc_gather(t4, pos_enc):
    mesh = plsc.VectorSubcoreMesh(core_axis_name="c", subcore_axis_name="s")

    @functools.partial(
        pl.kernel,
        out_type=jax.ShapeDtypeStruct((NS, D // 8, 32, 8, BT), jnp.float32),
        mesh=mesh,
        scratch_types=[
            pltpu.VMEM((NS, BT), jnp.int32),
            pltpu.VMEM((NBUF, BT, D), jnp.float32),
            pltpu.VMEM((NBUF, D, BT + 1), jnp.float32),
            pltpu.SemaphoreType.DMA,
            [pltpu.SemaphoreType.DMA] * NBUF,
            [pltpu.SemaphoreType.DMA] * NBUF,
        ],
        compiler_params=pltpu.CompilerParams(use_tc_tiling_on_sc=False,
                                             needs_layout_passes=False,
                                             disable_bounds_checks=True),
    )
    def k(t_hbm, table_hbm, out_hbm, idx_v, rows_v, tr_v,
          idx_sem, gat_sems, out_sems):
        wid = lax.axis_index("s") * 2 + lax.axis_index("c")
        iota = lax.broadcasted_iota(jnp.int32, (LANES,), 0)

        for rt in range(ST):
            pltpu.async_copy(t_hbm.at[rt, wid],
                             idx_v.at[pl.ds(rt * 8, 8)], idx_sem)
        for rt in range(ST):
            pltpu.make_async_copy(t_hbm.at[0, 0],
                                  idx_v.at[pl.ds(0, 8)], idx_sem).wait()

        def adj(r, carry):
            for c in range(BT // LANES):
                v = idx_v[r, pl.ds(c * LANES, LANES)]
                v = v - 1
                v = jnp.where(v < 0, MAXP - 1, v)
                idx_v[r, pl.ds(c * LANES, LANES)] = v
            return carry

        lax.fori_loop(0, NS, adj, 0)

        def fire_gather(s, slot):
            pltpu.async_copy(table_hbm.at[idx_v.at[s]], rows_v.at[slot],
                             gat_sems[slot])

        def drain_gather(slot):
            pltpu.make_async_copy(table_hbm.at[idx_v.at[0]], rows_v.at[0],
                                  gat_sems[slot]).wait()

        d_idx = tuple(iota + k * LANES for k in range(D // LANES))

        def transpose(slot):
            @plsc.parallel_loop(0, BT, unroll=8)
            def tbody(b):
                bv = jnp.full((LANES,), b, jnp.int32)
                for k in range(D // LANES):
                    v = rows_v[slot, b, pl.ds(k * LANES, LANES)]
                    plsc.store_scatter(tr_v.at[slot], [d_idx[k], bv], v)

        def fire_out(s, slot):
            for dt in range(D // 8):
                pltpu.async_copy(
                    tr_v.at[slot, pl.ds(dt * 8, 8), pl.ds(0, BT)],
                    out_hbm.at[s, dt, wid],
                    out_sems[slot])

        def drain_out(slot):
            for dt in range(D // 8):
                pltpu.make_async_copy(
                    tr_v.at[0, pl.ds(0, 8), pl.ds(0, BT)],
                    out_hbm.at[0, 0, 0],
                    out_sems[slot]).wait()

        def step(s, slot, do_fire, do_drain_out):
            if do_drain_out:
                drain_out(slot)
            if do_fire:
                fire_gather(s + AHEAD, (slot + AHEAD) % NBUF)
            drain_gather(slot)
            transpose(slot)
            fire_out(s, slot)

        for s in range(AHEAD):
            fire_gather(s, s % NBUF)

        for s in range(NBUF):
            step(s, s % NBUF, s + AHEAD < NS, False)

        groups = (NS - 2 * NBUF) // NBUF

        def outer(i, carry):
            for b in range(NBUF):
                step(i * NBUF + NBUF + b, b, True, True)
            return carry

        lax.fori_loop(0, groups, outer, 0)

        for s in range(NBUF + groups * NBUF, NS):
            step(s, s % NBUF, s + AHEAD < NS, True)

        for s in range(NS - NBUF, NS):
            drain_out(s % NBUF)

    return k(t4, pos_enc)


def kernel(t, pos_enc):
    t4 = t.T.reshape(ST, 8, 32, BT).transpose(0, 2, 1, 3)
    out4 = _sc_gather(t4, pos_enc)
    out5 = out4.reshape(NS, D // 8, 32, 8, BT)
    return out5.transpose(2, 4, 0, 1, 3).reshape(4096, NS, D)
